# baseline (speedup 1.0000x reference)
import jax
import jax.numpy as jnp
from jax.experimental import pallas as pl

_N_NODES = 10000
_D_FEAT = 128
_BLOCK_ROWS = 1000


def _copy_body(x_ref, o_ref):
    o_ref[...] = x_ref[...]


def kernel(edge_index, r_2, batch, x):
    return pl.pallas_call(
        _copy_body,
        out_shape=jax.ShapeDtypeStruct((_N_NODES, _D_FEAT), jnp.float32),
        grid=(_N_NODES // _BLOCK_ROWS,),
        in_specs=[pl.BlockSpec((_BLOCK_ROWS, _D_FEAT), lambda i: (i, 0))],
        out_specs=pl.BlockSpec((_BLOCK_ROWS, _D_FEAT), lambda i: (i, 0)),
    )(x)
